# Initial kernel scaffold; baseline (speedup 1.0000x reference)
import jax
import jax.numpy as jnp
from jax.experimental import pallas as pl


def kernel(u, w, h, d):
    raise NotImplementedError("write your pallas kernel here")


# speedup vs baseline: 4900.9725x; 4900.9725x over previous
import functools

import jax
import jax.numpy as jnp
from jax.experimental import pallas as pl
from jax.experimental.pallas import tpu as pltpu

_B = 3.0
_K = 5
_DIM = 128
_BR = 1024


def _rqs_block(u, wt, ht, dt):
    ew = jnp.exp(wt - jnp.max(wt, axis=0, keepdims=True))
    widths = ew / jnp.sum(ew, axis=0, keepdims=True) * (2.0 * _B)
    eh = jnp.exp(ht - jnp.max(ht, axis=0, keepdims=True))
    heights = eh / jnp.sum(eh, axis=0, keepdims=True) * (2.0 * _B)
    sp = jnp.maximum(dt, 0.0) + jnp.log1p(jnp.exp(-jnp.abs(dt)))

    one = jnp.ones((_DIM,), dtype=u.dtype)
    cw = [jnp.full((_DIM,), -_B, u.dtype)]
    ch = [jnp.full((_DIM,), -_B, u.dtype)]
    for k in range(_K - 1):
        cw.append(cw[-1] + widths[k])
        ch.append(ch[-1] + heights[k])
    cw.append(jnp.full((_DIM,), _B, u.dtype))
    ch.append(jnp.full((_DIM,), _B, u.dtype))
    wd = [widths[k] for k in range(_K)]
    hh = [heights[k] for k in range(_K)]
    dv = [one] + [sp[k] for k in range(_K - 1)] + [one]

    inside = (u >= -_B) & (u <= _B)
    uc = jnp.clip(u, -_B, _B)

    xk, wk, yk, hk = cw[0], wd[0], ch[0], hh[0]
    dk, dk1 = dv[0], dv[1]
    xk = jnp.broadcast_to(xk, u.shape)
    wk = jnp.broadcast_to(wk, u.shape)
    yk = jnp.broadcast_to(yk, u.shape)
    hk = jnp.broadcast_to(hk, u.shape)
    dk = jnp.broadcast_to(dk, u.shape)
    dk1 = jnp.broadcast_to(dk1, u.shape)
    for k in range(1, _K):
        m = uc >= cw[k]
        xk = jnp.where(m, cw[k], xk)
        wk = jnp.where(m, wd[k], wk)
        yk = jnp.where(m, ch[k], yk)
        hk = jnp.where(m, hh[k], hk)
        dk = jnp.where(m, dv[k], dk)
        dk1 = jnp.where(m, dv[k + 1], dk1)

    s = hk / wk
    theta = (uc - xk) / wk
    t1m = theta * (1.0 - theta)
    denom = s + (dk1 + dk - 2.0 * s) * t1m
    x_in = yk + hk * (s * theta * theta + dk * t1m) / denom
    num = s * s * (dk1 * theta * theta + 2.0 * s * t1m + dk * (1.0 - theta) ** 2)
    logd_in = jnp.log(num / (denom * denom))
    x = jnp.where(inside, x_in, u)
    ld = jnp.where(inside, logd_in, 0.0)
    return x, jnp.sum(ld, axis=-1, keepdims=True)


def _tc_body(u_ref, wt_ref, ht_ref, dt_ref, x_ref, ld_ref):
    x, ld = _rqs_block(u_ref[...], wt_ref[...], ht_ref[...], dt_ref[...])
    x_ref[...] = x
    ld_ref[...] = ld


def _tc_call(u, wt, ht, dt, interpret=False):
    n = u.shape[0]
    grid = (n // _BR,)
    x, ld = pl.pallas_call(
        _tc_body,
        grid=grid,
        in_specs=[
            pl.BlockSpec((_BR, _DIM), lambda i: (i, 0)),
            pl.BlockSpec((_K, _DIM), lambda i: (0, 0)),
            pl.BlockSpec((_K, _DIM), lambda i: (0, 0)),
            pl.BlockSpec((_K - 1, _DIM), lambda i: (0, 0)),
        ],
        out_specs=[
            pl.BlockSpec((_BR, _DIM), lambda i: (i, 0)),
            pl.BlockSpec((_BR, 1), lambda i: (i, 0)),
        ],
        out_shape=[
            jax.ShapeDtypeStruct((n, _DIM), u.dtype),
            jax.ShapeDtypeStruct((n, 1), u.dtype),
        ],
        interpret=interpret,
    )(u, wt, ht, dt)
    return x, ld.reshape(n)


@jax.jit
def kernel(u, w, h, d):
    wt = w.T
    ht = h.T
    dt = d.T
    return _tc_call(u, wt, ht, dt)
